# baseline (speedup 1.0000x reference)
import numpy as np
import jax
import jax.numpy as jnp
from jax.experimental import pallas as pl
from jax.experimental.pallas import tpu as pltpu

N = 10000
D_IN = 128
D_HID = 64
D_OUT = 40
P_DROP = 0.5
ROWS = 400
HALF = N // ROWS


def _make_scale():
    keep = jax.random.bernoulli(jax.random.key(42), 1.0 - P_DROP, (N, D_HID))
    return jnp.where(keep, 1.0 / (1.0 - P_DROP), 0.0).astype(jnp.float32)


try:
    with jax.default_device(jax.local_devices(backend="cpu")[0]):
        _SCALE = np.asarray(jax.jit(_make_scale)())
except Exception:
    _SCALE = np.asarray(_make_scale())


def _fused_body(adj_ref, x_hbm, w1_ref, b1_ref, scale_ref, w2_ref, b2_ref,
                out_ref, x_vmem, s1_vmem, s2_vmem, sem):
    i = pl.program_id(0)

    @pl.when(i == 0)
    def _():
        cp = pltpu.make_async_copy(x_hbm, x_vmem, sem)
        cp.start()
        cp.wait()
        s1_vmem[:] = jnp.dot(x_vmem[:], w1_ref[:],
                             preferred_element_type=jnp.float32)

    @pl.when(i < HALF)
    def _():
        m = jnp.dot(adj_ref[:], s1_vmem[:],
                    preferred_element_type=jnp.float32)
        m = jnp.maximum((m + b1_ref[:]) * scale_ref[:], 0.0)
        s2_vmem[pl.ds(i * ROWS, ROWS), :] = jnp.dot(
            m, w2_ref[:], preferred_element_type=jnp.float32)

    @pl.when(i >= HALF)
    def _():
        o = jnp.dot(adj_ref[:], s2_vmem[:],
                    preferred_element_type=jnp.float32)
        o = o + b2_ref[:]
        o = o - jnp.max(o, axis=1, keepdims=True)
        out_ref[:] = o - jnp.log(jnp.sum(jnp.exp(o), axis=1, keepdims=True))


def kernel(input, adj, W1, b1, W2, b2):
    x = input.astype(jnp.float32)
    scale = jnp.asarray(_SCALE)

    out = pl.pallas_call(
        _fused_body,
        grid=(2 * HALF,),
        in_specs=[
            pl.BlockSpec((ROWS, N), lambda i: (jax.lax.rem(i, HALF), 0)),
            pl.BlockSpec(memory_space=pl.ANY),
            pl.BlockSpec((D_IN, D_HID), lambda i: (0, 0)),
            pl.BlockSpec((1, D_HID), lambda i: (0, 0)),
            pl.BlockSpec((ROWS, D_HID),
                         lambda i: (jnp.where(i < HALF, i, HALF - 1), 0)),
            pl.BlockSpec((D_HID, D_OUT), lambda i: (0, 0)),
            pl.BlockSpec((1, D_OUT), lambda i: (0, 0)),
        ],
        out_specs=pl.BlockSpec(
            (ROWS, D_OUT), lambda i: (jnp.where(i < HALF, 0, i - HALF), 0)),
        out_shape=jax.ShapeDtypeStruct((N, D_OUT), jnp.float32),
        scratch_shapes=[
            pltpu.VMEM((N, D_IN), jnp.float32),
            pltpu.VMEM((N, D_HID), jnp.float32),
            pltpu.VMEM((N, D_OUT), jnp.float32),
            pltpu.SemaphoreType.DMA,
        ],
        compiler_params=pltpu.CompilerParams(
            dimension_semantics=("arbitrary",)),
    )(adj, x, W1, b1.reshape(1, D_HID), scale, W2, b2.reshape(1, D_OUT))
    return out
